# baseline (speedup 1.0000x reference)
import functools

import jax
import jax.numpy as jnp
from jax import lax
from jax.experimental import pallas as pl
from jax.experimental.pallas import tpu as pltpu
from jax.experimental.pallas import tpu_sc as plsc

_NC = 2
_NS = 16
_NW = _NC * _NS

_H = 128


_NBUF = 4


def _sc_gather_body(idx_hbm, tab_hbm, out_hbm, idx_v, rows_v, gsem, wsem, *,
                    per_w, chunk):
  wid = lax.axis_index("s") * _NC + lax.axis_index("c")
  base = wid * per_w
  nit = per_w // chunk

  pltpu.sync_copy(idx_hbm.at[pl.ds(base, per_w)], idx_v)

  def g_start(i, b):
    pltpu.async_copy(tab_hbm.at[idx_v.at[pl.ds(i * chunk, chunk)]],
                     rows_v.at[b], gsem.at[b])

  def g_wait(i, b):
    pltpu.make_async_copy(tab_hbm.at[idx_v.at[pl.ds(i * chunk, chunk)]],
                          rows_v.at[b], gsem.at[b]).wait()

  def w_start(i, b):
    pltpu.async_copy(rows_v.at[b],
                     out_hbm.at[pl.ds(base + i * chunk, chunk)], wsem.at[b])

  def w_wait(i, b):
    pltpu.make_async_copy(rows_v.at[b],
                          out_hbm.at[pl.ds(base + i * chunk, chunk)],
                          wsem.at[b]).wait()

  def step(i, b):
    if not isinstance(i, int) or i >= _NBUF:
      w_wait(i - _NBUF, b)
    g_start(i, b)
    prev = i - 2
    if not isinstance(prev, int) or prev >= 0:
      g_wait(prev, (b + 2) % _NBUF)
      w_start(prev, (b + 2) % _NBUF)

  n_quads = max(0, (nit - (2 * _NBUF - 2)) // _NBUF)
  i0 = nit - _NBUF * n_quads

  for i in range(i0):
    b = i % _NBUF
    if i >= _NBUF:
      w_wait(i - _NBUF, b)
    g_start(i, b)
    if i >= 2:
      g_wait(i - 2, (i - 2) % _NBUF)
      w_start(i - 2, (i - 2) % _NBUF)

  if n_quads > 0:
    def body(q, _):
      for r in range(_NBUF):
        step(i0 + q * _NBUF + r, (i0 + r) % _NBUF)
      return ()
    lax.fori_loop(0, n_quads, body, ())

  for i in range(nit - 2, nit):
    g_wait(i, i % _NBUF)
    w_start(i, i % _NBUF)
  for i in range(max(0, nit - _NBUF), nit):
    w_wait(i, i % _NBUF)


def _make_sc_gather(m, chunk, w, dtype):
  per_w = m // _NW
  mesh = plsc.VectorSubcoreMesh(core_axis_name="c", subcore_axis_name="s")
  body = functools.partial(_sc_gather_body, per_w=per_w, chunk=chunk)
  return pl.kernel(
      body,
      out_type=jax.ShapeDtypeStruct((m, w), dtype),
      mesh=mesh,
      scratch_types=[
          pltpu.VMEM((per_w,), jnp.int32),
          pltpu.VMEM((_NBUF, chunk, w), dtype),
          pltpu.SemaphoreType.DMA((_NBUF,)),
          pltpu.SemaphoreType.DMA((_NBUF,)),
      ],
  )


def _pack_hu(hv, uv):
  hb = lax.bitcast_convert_type(hv.astype(jnp.bfloat16), jnp.uint16)
  ub = lax.bitcast_convert_type(uv.astype(jnp.bfloat16), jnp.uint16)
  w = (ub.astype(jnp.uint32) << 16) | hb.astype(jnp.uint32)
  return lax.bitcast_convert_type(w, jnp.int32)


def _unpack_hu(w):
  wu = lax.bitcast_convert_type(w, jnp.uint32)
  hb = (wu & jnp.uint32(0xFFFF)).astype(jnp.uint16)
  ub = (wu >> 16).astype(jnp.uint16)
  hv = lax.bitcast_convert_type(hb, jnp.bfloat16).astype(jnp.float32)
  uv = lax.bitcast_convert_type(ub, jnp.bfloat16).astype(jnp.float32)
  return hv, uv


def _pre_body(f_ref, wz1_ref, wh1_ref, wr_ref, ur_ref, bz_ref, bh_ref, br_ref,
              fzr_ref, fh_ref, hu_ref, *, be):
  f = f_ref[...]
  fz = jnp.dot(f, wz1_ref[...], preferred_element_type=jnp.float32) + bz_ref[...]
  fh = jnp.dot(f, wh1_ref[...], preferred_element_type=jnp.float32) + bh_ref[...]
  r1 = jnp.dot(f, wr_ref[...], preferred_element_type=jnp.float32) + br_ref[...]
  h1 = jax.nn.sigmoid(fz) * jnp.tanh(fh)
  rows = lax.broadcasted_iota(jnp.int32, h1.shape, 0) + pl.program_id(0) * be
  h1 = jnp.where(rows > 0, h1, 0.0)
  u1 = jnp.dot(h1, ur_ref[...], preferred_element_type=jnp.float32)
  fzr_ref[...] = _pack_hu(fz, r1)
  fh_ref[...] = fh
  hu_ref[...] = _pack_hu(h1, u1)


def _gru_h(hun_ref, fzr_ref, fh_ref, wz2_ref, wh2_ref, be, off=0):
  hn, un = _unpack_hu(hun_ref[...])
  fz, r1 = _unpack_hu(fzr_ref[...])
  s_h = jnp.sum(hn, axis=0)
  r = jax.nn.sigmoid(r1[None, :, :] + un)
  s_g = jnp.sum(r * hn, axis=0)
  z = jax.nn.sigmoid(
      fz + jnp.dot(s_h.astype(jnp.bfloat16), wz2_ref[...],
                   preferred_element_type=jnp.float32))
  pre = jnp.tanh(
      fh_ref[...] +
      jnp.dot(s_g.astype(jnp.bfloat16), wh2_ref[...],
              preferred_element_type=jnp.float32))
  h = (1.0 - z) * s_h + z * pre
  rows = (lax.broadcasted_iota(jnp.int32, h.shape, 0)
          + (pl.program_id(0) + off) * be)
  return jnp.where(rows > 0, h, 0.0)


def _gate_mid_body(hun_ref, fzr_ref, fh_ref, wz2_ref, wh2_ref,
                   ur_ref, hu_ref, *, be, off=0):
  h = _gru_h(hun_ref, fzr_ref, fh_ref, wz2_ref, wh2_ref, be, off)
  hb = h.astype(jnp.bfloat16)
  u = jnp.dot(hb, ur_ref[...], preferred_element_type=jnp.float32)
  hu_ref[...] = _pack_hu(h, u)


def _gate_mid_alias_body(acc_ref, hun_ref, fzr_ref, fh_ref, wz2_ref,
                         wh2_ref, ur_ref, hu_ref, *, be, off=0):
  del acc_ref
  _gate_mid_body(hun_ref, fzr_ref, fh_ref, wz2_ref, wh2_ref,
                 ur_ref, hu_ref, be=be, off=off)


def _gate_last_body(hun_ref, fzr_ref, fh_ref, wz2_ref, wh2_ref,
                    h_ref, *, be, off=0):
  h_ref[...] = _gru_h(hun_ref, fzr_ref, fh_ref, wz2_ref, wh2_ref, be, off)


def _gate_last_alias_body(acc_ref, hun_ref, fzr_ref, fh_ref, wz2_ref,
                          wh2_ref, h_ref, *, be, off=0):
  del acc_ref
  _gate_last_body(hun_ref, fzr_ref, fh_ref, wz2_ref, wh2_ref,
                  h_ref, be=be, off=off)


def _out_body(hg_ref, fn_ref, msk_ref, wo1_ref, wo2_ref, bo_ref, o_ref):
  ns = jnp.sum(hg_ref[...], axis=0)
  o = jnp.dot(fn_ref[...], wo1_ref[...], preferred_element_type=jnp.float32)
  o += jnp.dot(ns, wo2_ref[...], preferred_element_type=jnp.float32)
  o += bo_ref[...]
  o_ref[...] = jnp.maximum(o, 0.0) * msk_ref[...]


def _row_spec(b, h, n_extra_lead=0):
  if n_extra_lead:
    return pl.BlockSpec((n_extra_lead, b, h), lambda i: (0, i, 0))
  return pl.BlockSpec((b, h), lambda i: (i, 0))


def _full_spec(shape):
  nd = len(shape)
  return pl.BlockSpec(shape, lambda i: (0,) * nd)


def kernel(fnode, fmess, agraph, bgraph, mask,
           Wz_w, Wz_b, Wr_w, Ur_w, Ur_b, Wh_w, Wh_b, Wo_w, Wo_b):
  n, fdim = fnode.shape
  e, h = fmess.shape
  nb = bgraph.shape[1]
  depth = 3

  Wz1, Wz2 = Wz_w[:h], Wz_w[h:]
  Wh1, Wh2 = Wh_w[:h], Wh_w[h:]
  Wo1, Wo2 = Wo_w[:fdim], Wo_w[fdim:]
  bz = Wz_b.reshape(1, h)
  bh = Wh_b.reshape(1, h)
  br = Ur_b.reshape(1, h)
  bo = Wo_b.reshape(1, h)

  n_slab = 5
  es = e // n_slab
  bidx_s = [bgraph[s * es:(s + 1) * es].T.reshape(-1) for s in range(n_slab)]
  npad = 10240
  ag = jnp.pad(agraph, ((0, npad - n), (0, 0)))
  aidx = ag.T.reshape(-1)
  fnode_p = jnp.pad(fnode, ((0, npad - n), (0, 0)))
  mask_p = jnp.pad(mask, ((0, npad - n), (0, 0)))

  be_pre = 1600
  g_pre = e // be_pre
  pre = pl.pallas_call(
      functools.partial(_pre_body, be=be_pre),
      grid=(g_pre,),
      in_specs=[
          _row_spec(be_pre, h),
          _full_spec((h, h)), _full_spec((h, h)), _full_spec((h, h)),
          _full_spec((h, h)),
          _full_spec((1, h)), _full_spec((1, h)), _full_spec((1, h)),
      ],
      out_specs=[_row_spec(be_pre, h)] * 3,
      out_shape=[jax.ShapeDtypeStruct((e, h), jnp.int32),
                 jax.ShapeDtypeStruct((e, h), jnp.float32),
                 jax.ShapeDtypeStruct((e, h), jnp.int32)],
      compiler_params=pltpu.CompilerParams(
          dimension_semantics=("arbitrary",)),
  )
  fzr, fhb, hu = pre(fmess, Wz1, Wh1, Wr_w, Ur_w, bz, bh, br)
  Wz2b = Wz2.astype(jnp.bfloat16)
  Wh2b = Wh2.astype(jnp.bfloat16)
  Urb = Ur_w.astype(jnp.bfloat16)

  sc_gather_hu = _make_sc_gather(nb * es, 120, h, jnp.int32)

  be_g = 640
  blk_slab = es // be_g

  def _mk_gate(body, out_dtype, n_weights, slab):
    off = slab * blk_slab
    aliased = slab > 0
    slab_spec = pl.BlockSpec((be_g, h), lambda i, off=off: (off + i, 0))
    in_specs = [
        _row_spec(be_g, h, nb),
        slab_spec, slab_spec,
    ] + [_full_spec((h, h))] * n_weights
    if aliased:
      in_specs = [pl.BlockSpec(memory_space=pl.ANY)] + in_specs
    return pl.pallas_call(
        functools.partial(body, be=be_g, off=off),
        grid=(blk_slab,),
        in_specs=in_specs,
        out_specs=slab_spec,
        out_shape=jax.ShapeDtypeStruct((e, h), out_dtype),
        input_output_aliases={0: 0} if aliased else {},
        compiler_params=pltpu.CompilerParams(
            dimension_semantics=("arbitrary",)),
    )

  gate_mid = [_mk_gate(_gate_mid_body if s == 0 else _gate_mid_alias_body,
                       jnp.int32, 3, s) for s in range(n_slab)]
  gate_last = [_mk_gate(_gate_last_body if s == 0 else _gate_last_alias_body,
                        jnp.float32, 2, s) for s in range(n_slab)]

  for d in range(depth - 1):
    huns = [sc_gather_hu(bidx_s[s], hu).reshape(nb, es, h)
            for s in range(n_slab)]
    last = d == depth - 2
    if not last:
      acc = gate_mid[0](huns[0], fzr, fhb, Wz2b, Wh2b, Urb)
      for s in range(1, n_slab):
        acc = gate_mid[s](acc, huns[s], fzr, fhb, Wz2b, Wh2b, Urb)
      hu = acc
    else:
      acc = gate_last[0](huns[0], fzr, fhb, Wz2b, Wh2b)
      for s in range(1, n_slab):
        acc = gate_last[s](acc, huns[s], fzr, fhb, Wz2b, Wh2b)
      hcur = acc

  sc_gather_h = _make_sc_gather(nb * npad, 120, h, jnp.float32)
  hg = sc_gather_h(aidx, hcur).reshape(nb, npad, h)

  bn = 1024
  g_o = npad // bn
  out = pl.pallas_call(
      _out_body,
      grid=(g_o,),
      in_specs=[
          _row_spec(bn, h, nb),
          _row_spec(bn, h),
          pl.BlockSpec((bn, 1), lambda i: (i, 0)),
          _full_spec((h, h)), _full_spec((h, h)), _full_spec((1, h)),
      ],
      out_specs=_row_spec(bn, h),
      out_shape=jax.ShapeDtypeStruct((npad, h), jnp.float32),
      compiler_params=pltpu.CompilerParams(
          dimension_semantics=("arbitrary",)),
  )
  node_h = out(hg, fnode_p, mask_p, Wo1, Wo2, bo)[:n]
  return node_h, hcur


# speedup vs baseline: 1.0025x; 1.0025x over previous
import functools

import jax
import jax.numpy as jnp
from jax import lax
from jax.experimental import pallas as pl
from jax.experimental.pallas import tpu as pltpu
from jax.experimental.pallas import tpu_sc as plsc

_NC = 2
_NS = 16
_NW = _NC * _NS

_H = 128


_NBUF = 4


def _sc_gather_body(idx_hbm, tab_hbm, out_hbm, idx_v, rows_v, gsem, wsem, *,
                    per_w, chunk):
  wid = lax.axis_index("s") * _NC + lax.axis_index("c")
  base = wid * per_w
  nit = per_w // chunk

  pltpu.sync_copy(idx_hbm.at[pl.ds(base, per_w)], idx_v)

  def g_start(i, b):
    pltpu.async_copy(tab_hbm.at[idx_v.at[pl.ds(i * chunk, chunk)]],
                     rows_v.at[b], gsem.at[b])

  def g_wait(i, b):
    pltpu.make_async_copy(tab_hbm.at[idx_v.at[pl.ds(i * chunk, chunk)]],
                          rows_v.at[b], gsem.at[b]).wait()

  def w_start(i, b):
    pltpu.async_copy(rows_v.at[b],
                     out_hbm.at[pl.ds(base + i * chunk, chunk)], wsem.at[b])

  def w_wait(i, b):
    pltpu.make_async_copy(rows_v.at[b],
                          out_hbm.at[pl.ds(base + i * chunk, chunk)],
                          wsem.at[b]).wait()

  def step(i, b):
    if not isinstance(i, int) or i >= _NBUF:
      w_wait(i - _NBUF, b)
    g_start(i, b)
    prev = i - 2
    if not isinstance(prev, int) or prev >= 0:
      g_wait(prev, (b + 2) % _NBUF)
      w_start(prev, (b + 2) % _NBUF)

  n_quads = max(0, (nit - (2 * _NBUF - 2)) // _NBUF)
  i0 = nit - _NBUF * n_quads

  for i in range(i0):
    b = i % _NBUF
    if i >= _NBUF:
      w_wait(i - _NBUF, b)
    g_start(i, b)
    if i >= 2:
      g_wait(i - 2, (i - 2) % _NBUF)
      w_start(i - 2, (i - 2) % _NBUF)

  if n_quads > 0:
    def body(q, _):
      for r in range(_NBUF):
        step(i0 + q * _NBUF + r, (i0 + r) % _NBUF)
      return ()
    lax.fori_loop(0, n_quads, body, ())

  for i in range(nit - 2, nit):
    g_wait(i, i % _NBUF)
    w_start(i, i % _NBUF)
  for i in range(max(0, nit - _NBUF), nit):
    w_wait(i, i % _NBUF)


def _make_sc_gather(m, chunk, w, dtype):
  per_w = m // _NW
  mesh = plsc.VectorSubcoreMesh(core_axis_name="c", subcore_axis_name="s")
  body = functools.partial(_sc_gather_body, per_w=per_w, chunk=chunk)
  return pl.kernel(
      body,
      out_type=jax.ShapeDtypeStruct((m, w), dtype),
      mesh=mesh,
      scratch_types=[
          pltpu.VMEM((per_w,), jnp.int32),
          pltpu.VMEM((_NBUF, chunk, w), dtype),
          pltpu.SemaphoreType.DMA((_NBUF,)),
          pltpu.SemaphoreType.DMA((_NBUF,)),
      ],
  )


def _pack_hu(hv, uv):
  hb = lax.bitcast_convert_type(hv.astype(jnp.bfloat16), jnp.uint16)
  ub = lax.bitcast_convert_type(uv.astype(jnp.bfloat16), jnp.uint16)
  w = (ub.astype(jnp.uint32) << 16) | hb.astype(jnp.uint32)
  return lax.bitcast_convert_type(w, jnp.int32)


def _unpack_hu(w):
  wu = lax.bitcast_convert_type(w, jnp.uint32)
  hb = (wu & jnp.uint32(0xFFFF)).astype(jnp.uint16)
  ub = (wu >> 16).astype(jnp.uint16)
  hv = lax.bitcast_convert_type(hb, jnp.bfloat16).astype(jnp.float32)
  uv = lax.bitcast_convert_type(ub, jnp.bfloat16).astype(jnp.float32)
  return hv, uv


def _pre_body(f_ref, wz1_ref, wh1_ref, wr_ref, ur_ref, bz_ref, bh_ref, br_ref,
              fzr_ref, fh_ref, hu_ref, *, be):
  f = f_ref[...]
  fz = jnp.dot(f, wz1_ref[...], preferred_element_type=jnp.float32) + bz_ref[...]
  fh = jnp.dot(f, wh1_ref[...], preferred_element_type=jnp.float32) + bh_ref[...]
  r1 = jnp.dot(f, wr_ref[...], preferred_element_type=jnp.float32) + br_ref[...]
  h1 = jax.nn.sigmoid(fz) * jnp.tanh(fh)
  rows = lax.broadcasted_iota(jnp.int32, h1.shape, 0) + pl.program_id(0) * be
  h1 = jnp.where(rows > 0, h1, 0.0)
  u1 = jnp.dot(h1, ur_ref[...], preferred_element_type=jnp.float32)
  fzr_ref[...] = _pack_hu(fz, r1)
  fh_ref[...] = fh
  hu_ref[...] = _pack_hu(h1, u1)


def _gru_h(hun_ref, fzr_ref, fh_ref, wz2_ref, wh2_ref, be, off=0):
  hn, un = _unpack_hu(hun_ref[...])
  fz, r1 = _unpack_hu(fzr_ref[...])
  s_h = jnp.sum(hn, axis=0)
  r = jax.nn.sigmoid(r1[None, :, :] + un)
  s_g = jnp.sum(r * hn, axis=0)
  z = jax.nn.sigmoid(
      fz + jnp.dot(s_h.astype(jnp.bfloat16), wz2_ref[...],
                   preferred_element_type=jnp.float32))
  pre = jnp.tanh(
      fh_ref[...] +
      jnp.dot(s_g.astype(jnp.bfloat16), wh2_ref[...],
              preferred_element_type=jnp.float32))
  h = (1.0 - z) * s_h + z * pre
  rows = (lax.broadcasted_iota(jnp.int32, h.shape, 0)
          + (pl.program_id(0) + off) * be)
  return jnp.where(rows > 0, h, 0.0)


def _gate_mid_body(hun_ref, fzr_ref, fh_ref, wz2_ref, wh2_ref,
                   ur_ref, hu_ref, *, be, off=0):
  h = _gru_h(hun_ref, fzr_ref, fh_ref, wz2_ref, wh2_ref, be, off)
  hb = h.astype(jnp.bfloat16)
  u = jnp.dot(hb, ur_ref[...], preferred_element_type=jnp.float32)
  hu_ref[...] = _pack_hu(h, u)


def _gate_mid_alias_body(acc_ref, hun_ref, fzr_ref, fh_ref, wz2_ref,
                         wh2_ref, ur_ref, hu_ref, *, be, off=0):
  del acc_ref
  _gate_mid_body(hun_ref, fzr_ref, fh_ref, wz2_ref, wh2_ref,
                 ur_ref, hu_ref, be=be, off=off)


def _gate_last_body(hun_ref, fzr_ref, fh_ref, wz2_ref, wh2_ref,
                    h_ref, *, be, off=0):
  h_ref[...] = _gru_h(hun_ref, fzr_ref, fh_ref, wz2_ref, wh2_ref, be, off)


def _gate_last_alias_body(acc_ref, hun_ref, fzr_ref, fh_ref, wz2_ref,
                          wh2_ref, h_ref, *, be, off=0):
  del acc_ref
  _gate_last_body(hun_ref, fzr_ref, fh_ref, wz2_ref, wh2_ref,
                  h_ref, be=be, off=off)


def _out_body(hg_ref, fn_ref, msk_ref, wo1_ref, wo2_ref, bo_ref, o_ref):
  ns = jnp.sum(hg_ref[...], axis=0)
  o = jnp.dot(fn_ref[...], wo1_ref[...], preferred_element_type=jnp.float32)
  o += jnp.dot(ns, wo2_ref[...], preferred_element_type=jnp.float32)
  o += bo_ref[...]
  o_ref[...] = jnp.maximum(o, 0.0) * msk_ref[...]


def _out_alias_body(acc_ref, hg_ref, fn_ref, msk_ref, wo1_ref, wo2_ref,
                    bo_ref, o_ref):
  del acc_ref
  _out_body(hg_ref, fn_ref, msk_ref, wo1_ref, wo2_ref, bo_ref, o_ref)


def _row_spec(b, h, n_extra_lead=0):
  if n_extra_lead:
    return pl.BlockSpec((n_extra_lead, b, h), lambda i: (0, i, 0))
  return pl.BlockSpec((b, h), lambda i: (i, 0))


def _full_spec(shape):
  nd = len(shape)
  return pl.BlockSpec(shape, lambda i: (0,) * nd)


def kernel(fnode, fmess, agraph, bgraph, mask,
           Wz_w, Wz_b, Wr_w, Ur_w, Ur_b, Wh_w, Wh_b, Wo_w, Wo_b):
  n, fdim = fnode.shape
  e, h = fmess.shape
  nb = bgraph.shape[1]
  depth = 3

  Wz1, Wz2 = Wz_w[:h], Wz_w[h:]
  Wh1, Wh2 = Wh_w[:h], Wh_w[h:]
  Wo1, Wo2 = Wo_w[:fdim], Wo_w[fdim:]
  bz = Wz_b.reshape(1, h)
  bh = Wh_b.reshape(1, h)
  br = Ur_b.reshape(1, h)
  bo = Wo_b.reshape(1, h)

  n_slab = 5
  es = e // n_slab
  bidx_s = [bgraph[s * es:(s + 1) * es].T.reshape(-1) for s in range(n_slab)]
  npad = 10240
  ag = jnp.pad(agraph, ((0, npad - n), (0, 0)))
  aidx = ag.T.reshape(-1)
  fnode_p = jnp.pad(fnode, ((0, npad - n), (0, 0)))
  mask_p = jnp.pad(mask, ((0, npad - n), (0, 0)))

  be_pre = 1600
  g_pre = e // be_pre
  pre = pl.pallas_call(
      functools.partial(_pre_body, be=be_pre),
      grid=(g_pre,),
      in_specs=[
          _row_spec(be_pre, h),
          _full_spec((h, h)), _full_spec((h, h)), _full_spec((h, h)),
          _full_spec((h, h)),
          _full_spec((1, h)), _full_spec((1, h)), _full_spec((1, h)),
      ],
      out_specs=[_row_spec(be_pre, h)] * 3,
      out_shape=[jax.ShapeDtypeStruct((e, h), jnp.int32),
                 jax.ShapeDtypeStruct((e, h), jnp.float32),
                 jax.ShapeDtypeStruct((e, h), jnp.int32)],
      compiler_params=pltpu.CompilerParams(
          dimension_semantics=("arbitrary",)),
  )
  fzr, fhb, hu = pre(fmess, Wz1, Wh1, Wr_w, Ur_w, bz, bh, br)
  Wz2b = Wz2.astype(jnp.bfloat16)
  Wh2b = Wh2.astype(jnp.bfloat16)
  Urb = Ur_w.astype(jnp.bfloat16)

  sc_gather_hu = _make_sc_gather(nb * es, 120, h, jnp.int32)

  be_g = 640
  blk_slab = es // be_g

  def _mk_gate(body, out_dtype, n_weights, slab):
    off = slab * blk_slab
    aliased = slab > 0
    slab_spec = pl.BlockSpec((be_g, h), lambda i, off=off: (off + i, 0))
    in_specs = [
        _row_spec(be_g, h, nb),
        slab_spec, slab_spec,
    ] + [_full_spec((h, h))] * n_weights
    if aliased:
      in_specs = [pl.BlockSpec(memory_space=pl.ANY)] + in_specs
    return pl.pallas_call(
        functools.partial(body, be=be_g, off=off),
        grid=(blk_slab,),
        in_specs=in_specs,
        out_specs=slab_spec,
        out_shape=jax.ShapeDtypeStruct((e, h), out_dtype),
        input_output_aliases={0: 0} if aliased else {},
        compiler_params=pltpu.CompilerParams(
            dimension_semantics=("arbitrary",)),
    )

  gate_mid = [_mk_gate(_gate_mid_body if s == 0 else _gate_mid_alias_body,
                       jnp.int32, 3, s) for s in range(n_slab)]
  gate_last = [_mk_gate(_gate_last_body if s == 0 else _gate_last_alias_body,
                        jnp.float32, 2, s) for s in range(n_slab)]

  for d in range(depth - 1):
    huns = [sc_gather_hu(bidx_s[s], hu).reshape(nb, es, h)
            for s in range(n_slab)]
    last = d == depth - 2
    if not last:
      acc = gate_mid[0](huns[0], fzr, fhb, Wz2b, Wh2b, Urb)
      for s in range(1, n_slab):
        acc = gate_mid[s](acc, huns[s], fzr, fhb, Wz2b, Wh2b, Urb)
      hu = acc
    else:
      acc = gate_last[0](huns[0], fzr, fhb, Wz2b, Wh2b)
      for s in range(1, n_slab):
        acc = gate_last[s](acc, huns[s], fzr, fhb, Wz2b, Wh2b)
      hcur = acc

  n_oslab = 2
  nps = npad // n_oslab
  aidx_s = [ag[s * nps:(s + 1) * nps].T.reshape(-1) for s in range(n_oslab)]
  sc_gather_h = _make_sc_gather(nb * nps, 120, h, jnp.float32)
  hgs = [sc_gather_h(aidx_s[s], hcur).reshape(nb, nps, h)
         for s in range(n_oslab)]

  bn = 1024
  blk_oslab = nps // bn

  def _mk_out(slab):
    off = slab * blk_oslab
    aliased = slab > 0
    in_specs = [
        _row_spec(bn, h, nb),
        pl.BlockSpec((bn, h), lambda i, off=off: (off + i, 0)),
        pl.BlockSpec((bn, 1), lambda i, off=off: (off + i, 0)),
        _full_spec((h, h)), _full_spec((h, h)), _full_spec((1, h)),
    ]
    if aliased:
      in_specs = [pl.BlockSpec(memory_space=pl.ANY)] + in_specs
    return pl.pallas_call(
        _out_alias_body if aliased else _out_body,
        grid=(blk_oslab,),
        in_specs=in_specs,
        out_specs=pl.BlockSpec((bn, h), lambda i, off=off: (off + i, 0)),
        out_shape=jax.ShapeDtypeStruct((npad, h), jnp.float32),
        input_output_aliases={0: 0} if aliased else {},
        compiler_params=pltpu.CompilerParams(
            dimension_semantics=("arbitrary",)),
    )

  acc = _mk_out(0)(hgs[0], fnode_p, mask_p, Wo1, Wo2, bo)
  for s in range(1, n_oslab):
    acc = _mk_out(s)(acc, hgs[s], fnode_p, mask_p, Wo1, Wo2, bo)
  return acc[:n], hcur


# speedup vs baseline: 1.0578x; 1.0551x over previous
import functools

import jax
import jax.numpy as jnp
from jax import lax
from jax.experimental import pallas as pl
from jax.experimental.pallas import tpu as pltpu
from jax.experimental.pallas import tpu_sc as plsc

_NC = 2
_NS = 16
_NW = _NC * _NS

_H = 128


_NBUF = 4


def _sc_gather_body(idx_hbm, tab_hbm, out_hbm, idx_v, rows_v, gsem, wsem, *,
                    per_w, chunk):
  wid = lax.axis_index("s") * _NC + lax.axis_index("c")
  base = wid * per_w
  nit = per_w // chunk

  pltpu.sync_copy(idx_hbm.at[pl.ds(base, per_w)], idx_v)

  def g_start(i, b):
    pltpu.async_copy(tab_hbm.at[idx_v.at[pl.ds(i * chunk, chunk)]],
                     rows_v.at[b], gsem.at[b])

  def g_wait(i, b):
    pltpu.make_async_copy(tab_hbm.at[idx_v.at[pl.ds(i * chunk, chunk)]],
                          rows_v.at[b], gsem.at[b]).wait()

  def w_start(i, b):
    pltpu.async_copy(rows_v.at[b],
                     out_hbm.at[pl.ds(base + i * chunk, chunk)], wsem.at[b])

  def w_wait(i, b):
    pltpu.make_async_copy(rows_v.at[b],
                          out_hbm.at[pl.ds(base + i * chunk, chunk)],
                          wsem.at[b]).wait()

  def step(i, b):
    if not isinstance(i, int) or i >= _NBUF:
      w_wait(i - _NBUF, b)
    g_start(i, b)
    prev = i - 2
    if not isinstance(prev, int) or prev >= 0:
      g_wait(prev, (b + 2) % _NBUF)
      w_start(prev, (b + 2) % _NBUF)

  n_quads = max(0, (nit - (2 * _NBUF - 2)) // _NBUF)
  i0 = nit - _NBUF * n_quads

  for i in range(i0):
    b = i % _NBUF
    if i >= _NBUF:
      w_wait(i - _NBUF, b)
    g_start(i, b)
    if i >= 2:
      g_wait(i - 2, (i - 2) % _NBUF)
      w_start(i - 2, (i - 2) % _NBUF)

  if n_quads > 0:
    def body(q, _):
      for r in range(_NBUF):
        step(i0 + q * _NBUF + r, (i0 + r) % _NBUF)
      return ()
    lax.fori_loop(0, n_quads, body, ())

  for i in range(nit - 2, nit):
    g_wait(i, i % _NBUF)
    w_start(i, i % _NBUF)
  for i in range(max(0, nit - _NBUF), nit):
    w_wait(i, i % _NBUF)


def _make_sc_gather(m, chunk, w, dtype):
  per_w = m // _NW
  mesh = plsc.VectorSubcoreMesh(core_axis_name="c", subcore_axis_name="s")
  body = functools.partial(_sc_gather_body, per_w=per_w, chunk=chunk)
  return pl.kernel(
      body,
      out_type=jax.ShapeDtypeStruct((m, w), dtype),
      mesh=mesh,
      scratch_types=[
          pltpu.VMEM((per_w,), jnp.int32),
          pltpu.VMEM((_NBUF, chunk, w), dtype),
          pltpu.SemaphoreType.DMA((_NBUF,)),
          pltpu.SemaphoreType.DMA((_NBUF,)),
      ],
  )


def _pack_hu(hv, uv):
  hb = lax.bitcast_convert_type(hv.astype(jnp.bfloat16), jnp.uint16)
  ub = lax.bitcast_convert_type(uv.astype(jnp.bfloat16), jnp.uint16)
  w = (ub.astype(jnp.uint32) << 16) | hb.astype(jnp.uint32)
  return lax.bitcast_convert_type(w, jnp.int32)


def _unpack_hu(w):
  wu = lax.bitcast_convert_type(w, jnp.uint32)
  hb = (wu & jnp.uint32(0xFFFF)).astype(jnp.uint16)
  ub = (wu >> 16).astype(jnp.uint16)
  hv = lax.bitcast_convert_type(hb, jnp.bfloat16).astype(jnp.float32)
  uv = lax.bitcast_convert_type(ub, jnp.bfloat16).astype(jnp.float32)
  return hv, uv


def _pre_body(f_ref, wz1_ref, wh1_ref, wr_ref, ur_ref, bz_ref, bh_ref, br_ref,
              fzr_ref, fh_ref, hu_ref, *, be):
  f = f_ref[...]
  fz = jnp.dot(f, wz1_ref[...], preferred_element_type=jnp.float32) + bz_ref[...]
  fh = jnp.dot(f, wh1_ref[...], preferred_element_type=jnp.float32) + bh_ref[...]
  r1 = jnp.dot(f, wr_ref[...], preferred_element_type=jnp.float32) + br_ref[...]
  h1 = jax.nn.sigmoid(fz) * jnp.tanh(fh)
  rows = lax.broadcasted_iota(jnp.int32, h1.shape, 0) + pl.program_id(0) * be
  h1 = jnp.where(rows > 0, h1, 0.0)
  u1 = jnp.dot(h1, ur_ref[...], preferred_element_type=jnp.float32)
  fzr_ref[...] = _pack_hu(fz, r1)
  fh_ref[...] = fh
  hu_ref[...] = _pack_hu(h1, u1)


def _gru_h(hun_ref, fzr_ref, fh_ref, wz2_ref, wh2_ref, be, off=0):
  hn, un = _unpack_hu(hun_ref[...])
  fz, r1 = _unpack_hu(fzr_ref[...])
  s_h = jnp.sum(hn, axis=0)
  r = jax.nn.sigmoid(r1[None, :, :] + un)
  s_g = jnp.sum(r * hn, axis=0)
  z = jax.nn.sigmoid(
      fz + jnp.dot(s_h.astype(jnp.bfloat16), wz2_ref[...],
                   preferred_element_type=jnp.float32))
  pre = jnp.tanh(
      fh_ref[...] +
      jnp.dot(s_g.astype(jnp.bfloat16), wh2_ref[...],
              preferred_element_type=jnp.float32))
  h = (1.0 - z) * s_h + z * pre
  rows = (lax.broadcasted_iota(jnp.int32, h.shape, 0)
          + (pl.program_id(0) + off) * be)
  return jnp.where(rows > 0, h, 0.0)


def _gate_mid_body(hun_ref, fzr_ref, fh_ref, wz2_ref, wh2_ref,
                   ur_ref, hu_ref, *, be, off=0):
  h = _gru_h(hun_ref, fzr_ref, fh_ref, wz2_ref, wh2_ref, be, off)
  hb = h.astype(jnp.bfloat16)
  u = jnp.dot(hb, ur_ref[...], preferred_element_type=jnp.float32)
  hu_ref[...] = _pack_hu(h, u)


def _gate_mid_alias_body(acc_ref, hun_ref, fzr_ref, fh_ref, wz2_ref,
                         wh2_ref, ur_ref, hu_ref, *, be, off=0):
  del acc_ref
  _gate_mid_body(hun_ref, fzr_ref, fh_ref, wz2_ref, wh2_ref,
                 ur_ref, hu_ref, be=be, off=off)


def _gate_last_body(hun_ref, fzr_ref, fh_ref, wz2_ref, wh2_ref,
                    h_ref, *, be, off=0):
  h_ref[...] = _gru_h(hun_ref, fzr_ref, fh_ref, wz2_ref, wh2_ref, be, off)


def _gate_last_alias_body(acc_ref, hun_ref, fzr_ref, fh_ref, wz2_ref,
                          wh2_ref, h_ref, *, be, off=0):
  del acc_ref
  _gate_last_body(hun_ref, fzr_ref, fh_ref, wz2_ref, wh2_ref,
                  h_ref, be=be, off=off)


def _out_body(hg_ref, fn_ref, msk_ref, wo1_ref, wo2_ref, bo_ref, o_ref):
  ns = jnp.sum(hg_ref[...], axis=0)
  o = jnp.dot(fn_ref[...], wo1_ref[...], preferred_element_type=jnp.float32)
  o += jnp.dot(ns, wo2_ref[...], preferred_element_type=jnp.float32)
  o += bo_ref[...]
  o_ref[...] = jnp.maximum(o, 0.0) * msk_ref[...]


def _out_alias_body(acc_ref, hg_ref, fn_ref, msk_ref, wo1_ref, wo2_ref,
                    bo_ref, o_ref):
  del acc_ref
  _out_body(hg_ref, fn_ref, msk_ref, wo1_ref, wo2_ref, bo_ref, o_ref)


def _row_spec(b, h, n_extra_lead=0):
  if n_extra_lead:
    return pl.BlockSpec((n_extra_lead, b, h), lambda i: (0, i, 0))
  return pl.BlockSpec((b, h), lambda i: (i, 0))


def _full_spec(shape):
  nd = len(shape)
  return pl.BlockSpec(shape, lambda i: (0,) * nd)


def kernel(fnode, fmess, agraph, bgraph, mask,
           Wz_w, Wz_b, Wr_w, Ur_w, Ur_b, Wh_w, Wh_b, Wo_w, Wo_b):
  n, fdim = fnode.shape
  e, h = fmess.shape
  nb = bgraph.shape[1]
  depth = 3

  Wz1, Wz2 = Wz_w[:h], Wz_w[h:]
  Wh1, Wh2 = Wh_w[:h], Wh_w[h:]
  Wo1, Wo2 = Wo_w[:fdim], Wo_w[fdim:]
  bz = Wz_b.reshape(1, h)
  bh = Wh_b.reshape(1, h)
  br = Ur_b.reshape(1, h)
  bo = Wo_b.reshape(1, h)

  n_slab = 5
  es = e // n_slab
  bidx_s = [bgraph[s * es:(s + 1) * es].T.reshape(-1) for s in range(n_slab)]
  npad = 10240
  ag = jnp.pad(agraph, ((0, npad - n), (0, 0)))
  aidx = ag.T.reshape(-1)
  fnode_p = jnp.pad(fnode, ((0, npad - n), (0, 0)))
  mask_p = jnp.pad(mask, ((0, npad - n), (0, 0)))

  be_pre = 1600
  g_pre = e // be_pre
  pre = pl.pallas_call(
      functools.partial(_pre_body, be=be_pre),
      grid=(g_pre,),
      in_specs=[
          _row_spec(be_pre, h),
          _full_spec((h, h)), _full_spec((h, h)), _full_spec((h, h)),
          _full_spec((h, h)),
          _full_spec((1, h)), _full_spec((1, h)), _full_spec((1, h)),
      ],
      out_specs=[_row_spec(be_pre, h)] * 3,
      out_shape=[jax.ShapeDtypeStruct((e, h), jnp.int32),
                 jax.ShapeDtypeStruct((e, h), jnp.float32),
                 jax.ShapeDtypeStruct((e, h), jnp.int32)],
      compiler_params=pltpu.CompilerParams(
          dimension_semantics=("arbitrary",)),
  )
  fzr, fhb, hu = pre(fmess, Wz1, Wh1, Wr_w, Ur_w, bz, bh, br)
  Wz2b = Wz2.astype(jnp.bfloat16)
  Wh2b = Wh2.astype(jnp.bfloat16)
  Urb = Ur_w.astype(jnp.bfloat16)

  sc_gather_hu = _make_sc_gather(nb * es, 120, h, jnp.int32)

  be_g = 1280
  blk_slab = es // be_g

  def _mk_gate(body, out_dtype, n_weights, slab):
    off = slab * blk_slab
    aliased = slab > 0
    slab_spec = pl.BlockSpec((be_g, h), lambda i, off=off: (off + i, 0))
    in_specs = [
        _row_spec(be_g, h, nb),
        slab_spec, slab_spec,
    ] + [_full_spec((h, h))] * n_weights
    if aliased:
      in_specs = [pl.BlockSpec(memory_space=pl.ANY)] + in_specs
    return pl.pallas_call(
        functools.partial(body, be=be_g, off=off),
        grid=(blk_slab,),
        in_specs=in_specs,
        out_specs=slab_spec,
        out_shape=jax.ShapeDtypeStruct((e, h), out_dtype),
        input_output_aliases={0: 0} if aliased else {},
        compiler_params=pltpu.CompilerParams(
            dimension_semantics=("arbitrary",)),
    )

  gate_mid = [_mk_gate(_gate_mid_body if s == 0 else _gate_mid_alias_body,
                       jnp.int32, 3, s) for s in range(n_slab)]
  gate_last = [_mk_gate(_gate_last_body if s == 0 else _gate_last_alias_body,
                        jnp.float32, 2, s) for s in range(n_slab)]

  for d in range(depth - 1):
    huns = [sc_gather_hu(bidx_s[s], hu).reshape(nb, es, h)
            for s in range(n_slab)]
    last = d == depth - 2
    if not last:
      acc = gate_mid[0](huns[0], fzr, fhb, Wz2b, Wh2b, Urb)
      for s in range(1, n_slab):
        acc = gate_mid[s](acc, huns[s], fzr, fhb, Wz2b, Wh2b, Urb)
      hu = acc
    else:
      acc = gate_last[0](huns[0], fzr, fhb, Wz2b, Wh2b)
      for s in range(1, n_slab):
        acc = gate_last[s](acc, huns[s], fzr, fhb, Wz2b, Wh2b)
      hcur = acc

  n_oslab = 2
  nps = npad // n_oslab
  aidx_s = [ag[s * nps:(s + 1) * nps].T.reshape(-1) for s in range(n_oslab)]
  sc_gather_h = _make_sc_gather(nb * nps, 120, h, jnp.float32)
  hgs = [sc_gather_h(aidx_s[s], hcur).reshape(nb, nps, h)
         for s in range(n_oslab)]

  bn = 1024
  blk_oslab = nps // bn

  def _mk_out(slab):
    off = slab * blk_oslab
    aliased = slab > 0
    in_specs = [
        _row_spec(bn, h, nb),
        pl.BlockSpec((bn, h), lambda i, off=off: (off + i, 0)),
        pl.BlockSpec((bn, 1), lambda i, off=off: (off + i, 0)),
        _full_spec((h, h)), _full_spec((h, h)), _full_spec((1, h)),
    ]
    if aliased:
      in_specs = [pl.BlockSpec(memory_space=pl.ANY)] + in_specs
    return pl.pallas_call(
        _out_alias_body if aliased else _out_body,
        grid=(blk_oslab,),
        in_specs=in_specs,
        out_specs=pl.BlockSpec((bn, h), lambda i, off=off: (off + i, 0)),
        out_shape=jax.ShapeDtypeStruct((npad, h), jnp.float32),
        input_output_aliases={0: 0} if aliased else {},
        compiler_params=pltpu.CompilerParams(
            dimension_semantics=("arbitrary",)),
    )

  acc = _mk_out(0)(hgs[0], fnode_p, mask_p, Wo1, Wo2, bo)
  for s in range(1, n_oslab):
    acc = _mk_out(s)(acc, hgs[s], fnode_p, mask_p, Wo1, Wo2, bo)
  return acc[:n], hcur


# speedup vs baseline: 1.0631x; 1.0050x over previous
import functools

import jax
import jax.numpy as jnp
from jax import lax
from jax.experimental import pallas as pl
from jax.experimental.pallas import tpu as pltpu
from jax.experimental.pallas import tpu_sc as plsc

_NC = 2
_NS = 16
_NW = _NC * _NS

_H = 128


_NBUF = 4


def _sc_gather_body(idx_hbm, tab_hbm, out_hbm, idx_v, rows_v, gsem, wsem, *,
                    per_w, chunk):
  wid = lax.axis_index("s") * _NC + lax.axis_index("c")
  base = wid * per_w
  nit = per_w // chunk

  pltpu.sync_copy(idx_hbm.at[pl.ds(base, per_w)], idx_v)

  def g_start(i, b):
    pltpu.async_copy(tab_hbm.at[idx_v.at[pl.ds(i * chunk, chunk)]],
                     rows_v.at[b], gsem.at[b])

  def g_wait(i, b):
    pltpu.make_async_copy(tab_hbm.at[idx_v.at[pl.ds(i * chunk, chunk)]],
                          rows_v.at[b], gsem.at[b]).wait()

  def w_start(i, b):
    pltpu.async_copy(rows_v.at[b],
                     out_hbm.at[pl.ds(base + i * chunk, chunk)], wsem.at[b])

  def w_wait(i, b):
    pltpu.make_async_copy(rows_v.at[b],
                          out_hbm.at[pl.ds(base + i * chunk, chunk)],
                          wsem.at[b]).wait()

  def step(i, b):
    if not isinstance(i, int) or i >= _NBUF:
      w_wait(i - _NBUF, b)
    g_start(i, b)
    prev = i - 2
    if not isinstance(prev, int) or prev >= 0:
      g_wait(prev, (b + 2) % _NBUF)
      w_start(prev, (b + 2) % _NBUF)

  n_quads = max(0, (nit - (2 * _NBUF - 2)) // _NBUF)
  i0 = nit - _NBUF * n_quads

  for i in range(i0):
    b = i % _NBUF
    if i >= _NBUF:
      w_wait(i - _NBUF, b)
    g_start(i, b)
    if i >= 2:
      g_wait(i - 2, (i - 2) % _NBUF)
      w_start(i - 2, (i - 2) % _NBUF)

  if n_quads > 0:
    def body(q, _):
      for r in range(_NBUF):
        step(i0 + q * _NBUF + r, (i0 + r) % _NBUF)
      return ()
    lax.fori_loop(0, n_quads, body, ())

  for i in range(nit - 2, nit):
    g_wait(i, i % _NBUF)
    w_start(i, i % _NBUF)
  for i in range(max(0, nit - _NBUF), nit):
    w_wait(i, i % _NBUF)


def _make_sc_gather(m, chunk, w, dtype):
  per_w = m // _NW
  mesh = plsc.VectorSubcoreMesh(core_axis_name="c", subcore_axis_name="s")
  body = functools.partial(_sc_gather_body, per_w=per_w, chunk=chunk)
  return pl.kernel(
      body,
      out_type=jax.ShapeDtypeStruct((m, w), dtype),
      mesh=mesh,
      scratch_types=[
          pltpu.VMEM((per_w,), jnp.int32),
          pltpu.VMEM((_NBUF, chunk, w), dtype),
          pltpu.SemaphoreType.DMA((_NBUF,)),
          pltpu.SemaphoreType.DMA((_NBUF,)),
      ],
  )


def _pack_hu(hv, uv):
  hb = lax.bitcast_convert_type(hv.astype(jnp.bfloat16), jnp.uint16)
  ub = lax.bitcast_convert_type(uv.astype(jnp.bfloat16), jnp.uint16)
  w = (ub.astype(jnp.uint32) << 16) | hb.astype(jnp.uint32)
  return lax.bitcast_convert_type(w, jnp.int32)


def _unpack_hu(w):
  wu = lax.bitcast_convert_type(w, jnp.uint32)
  hb = (wu & jnp.uint32(0xFFFF)).astype(jnp.uint16)
  ub = (wu >> 16).astype(jnp.uint16)
  hv = lax.bitcast_convert_type(hb, jnp.bfloat16).astype(jnp.float32)
  uv = lax.bitcast_convert_type(ub, jnp.bfloat16).astype(jnp.float32)
  return hv, uv


def _pre_body(f_ref, wz1_ref, wh1_ref, wr_ref, ur_ref, bz_ref, bh_ref, br_ref,
              fzr_ref, fh_ref, hu_ref, *, be):
  f = f_ref[...]
  fz = jnp.dot(f, wz1_ref[...], preferred_element_type=jnp.float32) + bz_ref[...]
  fh = jnp.dot(f, wh1_ref[...], preferred_element_type=jnp.float32) + bh_ref[...]
  r1 = jnp.dot(f, wr_ref[...], preferred_element_type=jnp.float32) + br_ref[...]
  h1 = jax.nn.sigmoid(fz) * jnp.tanh(fh)
  rows = lax.broadcasted_iota(jnp.int32, h1.shape, 0) + pl.program_id(0) * be
  h1 = jnp.where(rows > 0, h1, 0.0)
  u1 = jnp.dot(h1, ur_ref[...], preferred_element_type=jnp.float32)
  fzr_ref[...] = _pack_hu(fz, r1)
  fh_ref[...] = fh
  hu_ref[...] = _pack_hu(h1, u1)


def _gru_h(hun_ref, fzr_ref, fh_ref, wz2_ref, wh2_ref, be, off=0):
  hn, un = _unpack_hu(hun_ref[...])
  fz, r1 = _unpack_hu(fzr_ref[...])
  s_h = jnp.sum(hn, axis=0)
  r = jax.nn.sigmoid(r1[None, :, :] + un)
  s_g = jnp.sum(r * hn, axis=0)
  z = jax.nn.sigmoid(
      fz + jnp.dot(s_h.astype(jnp.bfloat16), wz2_ref[...],
                   preferred_element_type=jnp.float32))
  pre = jnp.tanh(
      fh_ref[...] +
      jnp.dot(s_g.astype(jnp.bfloat16), wh2_ref[...],
              preferred_element_type=jnp.float32))
  h = (1.0 - z) * s_h + z * pre
  rows = (lax.broadcasted_iota(jnp.int32, h.shape, 0)
          + (pl.program_id(0) + off) * be)
  return jnp.where(rows > 0, h, 0.0)


def _gate_mid_body(hun_ref, fzr_ref, fh_ref, wz2_ref, wh2_ref,
                   ur_ref, hu_ref, *, be, off=0):
  h = _gru_h(hun_ref, fzr_ref, fh_ref, wz2_ref, wh2_ref, be, off)
  hb = h.astype(jnp.bfloat16)
  u = jnp.dot(hb, ur_ref[...], preferred_element_type=jnp.float32)
  hu_ref[...] = _pack_hu(h, u)


def _gate_mid_alias_body(acc_ref, hun_ref, fzr_ref, fh_ref, wz2_ref,
                         wh2_ref, ur_ref, hu_ref, *, be, off=0):
  del acc_ref
  _gate_mid_body(hun_ref, fzr_ref, fh_ref, wz2_ref, wh2_ref,
                 ur_ref, hu_ref, be=be, off=off)


def _gate_last_body(hun_ref, fzr_ref, fh_ref, wz2_ref, wh2_ref,
                    h_ref, *, be, off=0):
  h_ref[...] = _gru_h(hun_ref, fzr_ref, fh_ref, wz2_ref, wh2_ref, be, off)


def _gate_last_alias_body(acc_ref, hun_ref, fzr_ref, fh_ref, wz2_ref,
                          wh2_ref, h_ref, *, be, off=0):
  del acc_ref
  _gate_last_body(hun_ref, fzr_ref, fh_ref, wz2_ref, wh2_ref,
                  h_ref, be=be, off=off)


def _out_body(hg_ref, fn_ref, msk_ref, wo1_ref, wo2_ref, bo_ref, o_ref):
  ns = jnp.sum(hg_ref[...], axis=0)
  o = jnp.dot(fn_ref[...], wo1_ref[...], preferred_element_type=jnp.float32)
  o += jnp.dot(ns, wo2_ref[...], preferred_element_type=jnp.float32)
  o += bo_ref[...]
  o_ref[...] = jnp.maximum(o, 0.0) * msk_ref[...]


def _out_alias_body(acc_ref, hg_ref, fn_ref, msk_ref, wo1_ref, wo2_ref,
                    bo_ref, o_ref):
  del acc_ref
  _out_body(hg_ref, fn_ref, msk_ref, wo1_ref, wo2_ref, bo_ref, o_ref)


def _row_spec(b, h, n_extra_lead=0):
  if n_extra_lead:
    return pl.BlockSpec((n_extra_lead, b, h), lambda i: (0, i, 0))
  return pl.BlockSpec((b, h), lambda i: (i, 0))


def _full_spec(shape):
  nd = len(shape)
  return pl.BlockSpec(shape, lambda i: (0,) * nd)


def kernel(fnode, fmess, agraph, bgraph, mask,
           Wz_w, Wz_b, Wr_w, Ur_w, Ur_b, Wh_w, Wh_b, Wo_w, Wo_b):
  n, fdim = fnode.shape
  e, h = fmess.shape
  nb = bgraph.shape[1]
  depth = 3

  Wz1, Wz2 = Wz_w[:h], Wz_w[h:]
  Wh1, Wh2 = Wh_w[:h], Wh_w[h:]
  Wo1, Wo2 = Wo_w[:fdim], Wo_w[fdim:]
  bz = Wz_b.reshape(1, h)
  bh = Wh_b.reshape(1, h)
  br = Ur_b.reshape(1, h)
  bo = Wo_b.reshape(1, h)

  n_slab = 5
  es = e // n_slab
  bidx_s = [bgraph[s * es:(s + 1) * es].T.reshape(-1) for s in range(n_slab)]
  npad = 10240
  ag = jnp.pad(agraph, ((0, npad - n), (0, 0)))
  aidx = ag.T.reshape(-1)
  fnode_p = jnp.pad(fnode, ((0, npad - n), (0, 0)))
  mask_p = jnp.pad(mask, ((0, npad - n), (0, 0)))

  be_pre = 1600
  g_pre = e // be_pre
  pre = pl.pallas_call(
      functools.partial(_pre_body, be=be_pre),
      grid=(g_pre,),
      in_specs=[
          _row_spec(be_pre, h),
          _full_spec((h, h)), _full_spec((h, h)), _full_spec((h, h)),
          _full_spec((h, h)),
          _full_spec((1, h)), _full_spec((1, h)), _full_spec((1, h)),
      ],
      out_specs=[_row_spec(be_pre, h)] * 3,
      out_shape=[jax.ShapeDtypeStruct((e, h), jnp.int32),
                 jax.ShapeDtypeStruct((e, h), jnp.float32),
                 jax.ShapeDtypeStruct((e, h), jnp.int32)],
      compiler_params=pltpu.CompilerParams(
          dimension_semantics=("arbitrary",)),
  )
  fzr, fhb, hu = pre(fmess, Wz1, Wh1, Wr_w, Ur_w, bz, bh, br)
  Wz2b = Wz2.astype(jnp.bfloat16)
  Wh2b = Wh2.astype(jnp.bfloat16)
  Urb = Ur_w.astype(jnp.bfloat16)

  sc_gather_hu = _make_sc_gather(nb * es, 120, h, jnp.int32)

  be_g = 1600
  blk_slab = es // be_g

  def _mk_gate(body, out_dtype, n_weights, slab):
    off = slab * blk_slab
    aliased = slab > 0
    slab_spec = pl.BlockSpec((be_g, h), lambda i, off=off: (off + i, 0))
    in_specs = [
        _row_spec(be_g, h, nb),
        slab_spec, slab_spec,
    ] + [_full_spec((h, h))] * n_weights
    if aliased:
      in_specs = [pl.BlockSpec(memory_space=pl.ANY)] + in_specs
    return pl.pallas_call(
        functools.partial(body, be=be_g, off=off),
        grid=(blk_slab,),
        in_specs=in_specs,
        out_specs=slab_spec,
        out_shape=jax.ShapeDtypeStruct((e, h), out_dtype),
        input_output_aliases={0: 0} if aliased else {},
        compiler_params=pltpu.CompilerParams(
            dimension_semantics=("arbitrary",)),
    )

  gate_mid = [_mk_gate(_gate_mid_body if s == 0 else _gate_mid_alias_body,
                       jnp.int32, 3, s) for s in range(n_slab)]
  gate_last = [_mk_gate(_gate_last_body if s == 0 else _gate_last_alias_body,
                        jnp.float32, 2, s) for s in range(n_slab)]

  for d in range(depth - 1):
    huns = [sc_gather_hu(bidx_s[s], hu).reshape(nb, es, h)
            for s in range(n_slab)]
    last = d == depth - 2
    if not last:
      acc = gate_mid[0](huns[0], fzr, fhb, Wz2b, Wh2b, Urb)
      for s in range(1, n_slab):
        acc = gate_mid[s](acc, huns[s], fzr, fhb, Wz2b, Wh2b, Urb)
      hu = acc
    else:
      acc = gate_last[0](huns[0], fzr, fhb, Wz2b, Wh2b)
      for s in range(1, n_slab):
        acc = gate_last[s](acc, huns[s], fzr, fhb, Wz2b, Wh2b)
      hcur = acc

  n_oslab = 2
  nps = npad // n_oslab
  aidx_s = [ag[s * nps:(s + 1) * nps].T.reshape(-1) for s in range(n_oslab)]
  sc_gather_h = _make_sc_gather(nb * nps, 120, h, jnp.float32)
  hgs = [sc_gather_h(aidx_s[s], hcur).reshape(nb, nps, h)
         for s in range(n_oslab)]

  bn = 1024
  blk_oslab = nps // bn

  def _mk_out(slab):
    off = slab * blk_oslab
    aliased = slab > 0
    in_specs = [
        _row_spec(bn, h, nb),
        pl.BlockSpec((bn, h), lambda i, off=off: (off + i, 0)),
        pl.BlockSpec((bn, 1), lambda i, off=off: (off + i, 0)),
        _full_spec((h, h)), _full_spec((h, h)), _full_spec((1, h)),
    ]
    if aliased:
      in_specs = [pl.BlockSpec(memory_space=pl.ANY)] + in_specs
    return pl.pallas_call(
        _out_alias_body if aliased else _out_body,
        grid=(blk_oslab,),
        in_specs=in_specs,
        out_specs=pl.BlockSpec((bn, h), lambda i, off=off: (off + i, 0)),
        out_shape=jax.ShapeDtypeStruct((npad, h), jnp.float32),
        input_output_aliases={0: 0} if aliased else {},
        compiler_params=pltpu.CompilerParams(
            dimension_semantics=("arbitrary",)),
    )

  acc = _mk_out(0)(hgs[0], fnode_p, mask_p, Wo1, Wo2, bo)
  for s in range(1, n_oslab):
    acc = _mk_out(s)(acc, hgs[s], fnode_p, mask_p, Wo1, Wo2, bo)
  return acc[:n], hcur


# speedup vs baseline: 1.0946x; 1.0296x over previous
import functools

import jax
import jax.numpy as jnp
from jax import lax
from jax.experimental import pallas as pl
from jax.experimental.pallas import tpu as pltpu
from jax.experimental.pallas import tpu_sc as plsc

_NC = 2
_NS = 16
_NW = _NC * _NS

_H = 128


_NBUF = 4


def _sc_gather_body(idx_hbm, tab_hbm, out_hbm, idx_v, rows_v, gsem, wsem, *,
                    per_w, chunk):
  wid = lax.axis_index("s") * _NC + lax.axis_index("c")
  base = wid * per_w
  nit = per_w // chunk

  pltpu.sync_copy(idx_hbm.at[pl.ds(base, per_w)], idx_v)

  def g_start(i, b):
    pltpu.async_copy(tab_hbm.at[idx_v.at[pl.ds(i * chunk, chunk)]],
                     rows_v.at[b], gsem.at[b])

  def g_wait(i, b):
    pltpu.make_async_copy(tab_hbm.at[idx_v.at[pl.ds(i * chunk, chunk)]],
                          rows_v.at[b], gsem.at[b]).wait()

  def w_start(i, b):
    pltpu.async_copy(rows_v.at[b],
                     out_hbm.at[pl.ds(base + i * chunk, chunk)], wsem.at[b])

  def w_wait(i, b):
    pltpu.make_async_copy(rows_v.at[b],
                          out_hbm.at[pl.ds(base + i * chunk, chunk)],
                          wsem.at[b]).wait()

  def step(i, b):
    if not isinstance(i, int) or i >= _NBUF:
      w_wait(i - _NBUF, b)
    g_start(i, b)
    prev = i - 2
    if not isinstance(prev, int) or prev >= 0:
      g_wait(prev, (b + 2) % _NBUF)
      w_start(prev, (b + 2) % _NBUF)

  n_quads = max(0, (nit - (2 * _NBUF - 2)) // _NBUF)
  i0 = nit - _NBUF * n_quads

  for i in range(i0):
    b = i % _NBUF
    if i >= _NBUF:
      w_wait(i - _NBUF, b)
    g_start(i, b)
    if i >= 2:
      g_wait(i - 2, (i - 2) % _NBUF)
      w_start(i - 2, (i - 2) % _NBUF)

  if n_quads > 0:
    def body(q, _):
      for r in range(_NBUF):
        step(i0 + q * _NBUF + r, (i0 + r) % _NBUF)
      return ()
    lax.fori_loop(0, n_quads, body, ())

  for i in range(nit - 2, nit):
    g_wait(i, i % _NBUF)
    w_start(i, i % _NBUF)
  for i in range(max(0, nit - _NBUF), nit):
    w_wait(i, i % _NBUF)


def _make_sc_gather(m, chunk, w, dtype):
  per_w = m // _NW
  mesh = plsc.VectorSubcoreMesh(core_axis_name="c", subcore_axis_name="s")
  body = functools.partial(_sc_gather_body, per_w=per_w, chunk=chunk)
  return pl.kernel(
      body,
      out_type=jax.ShapeDtypeStruct((m, w), dtype),
      mesh=mesh,
      scratch_types=[
          pltpu.VMEM((per_w,), jnp.int32),
          pltpu.VMEM((_NBUF, chunk, w), dtype),
          pltpu.SemaphoreType.DMA((_NBUF,)),
          pltpu.SemaphoreType.DMA((_NBUF,)),
      ],
  )


def _pack_hu(hv, uv):
  hb = lax.bitcast_convert_type(hv.astype(jnp.bfloat16), jnp.uint16)
  ub = lax.bitcast_convert_type(uv.astype(jnp.bfloat16), jnp.uint16)
  w = (ub.astype(jnp.uint32) << 16) | hb.astype(jnp.uint32)
  return lax.bitcast_convert_type(w, jnp.int32)


def _unpack_hu(w):
  wu = lax.bitcast_convert_type(w, jnp.uint32)
  hb = (wu & jnp.uint32(0xFFFF)).astype(jnp.uint16)
  ub = (wu >> 16).astype(jnp.uint16)
  hv = lax.bitcast_convert_type(hb, jnp.bfloat16).astype(jnp.float32)
  uv = lax.bitcast_convert_type(ub, jnp.bfloat16).astype(jnp.float32)
  return hv, uv


def _pre_body(f_ref, wz1_ref, wh1_ref, wr_ref, ur_ref, bz_ref, bh_ref, br_ref,
              fzr_ref, fh_ref, hu_ref, *, be):
  f = f_ref[...]
  fz = jnp.dot(f, wz1_ref[...], preferred_element_type=jnp.float32) + bz_ref[...]
  fh = jnp.dot(f, wh1_ref[...], preferred_element_type=jnp.float32) + bh_ref[...]
  r1 = jnp.dot(f, wr_ref[...], preferred_element_type=jnp.float32) + br_ref[...]
  h1 = jax.nn.sigmoid(fz) * jnp.tanh(fh)
  rows = lax.broadcasted_iota(jnp.int32, h1.shape, 0) + pl.program_id(0) * be
  h1 = jnp.where(rows > 0, h1, 0.0)
  u1 = jnp.dot(h1, ur_ref[...], preferred_element_type=jnp.float32)
  fzr_ref[...] = _pack_hu(fz, r1)
  fh_ref[...] = fh
  hu_ref[...] = _pack_hu(h1, u1)


def _gru_h(hun_ref, fzr_ref, fh_ref, wz2_ref, wh2_ref, be, off=0):
  hn, un = _unpack_hu(hun_ref[...])
  fz, r1 = _unpack_hu(fzr_ref[...])
  s_h = jnp.sum(hn, axis=0)
  r = jax.nn.sigmoid(r1[None, :, :] + un)
  s_g = jnp.sum(r * hn, axis=0)
  z = jax.nn.sigmoid(
      fz + jnp.dot(s_h.astype(jnp.bfloat16), wz2_ref[...],
                   preferred_element_type=jnp.float32))
  pre = jnp.tanh(
      fh_ref[...] +
      jnp.dot(s_g.astype(jnp.bfloat16), wh2_ref[...],
              preferred_element_type=jnp.float32))
  h = (1.0 - z) * s_h + z * pre
  rows = (lax.broadcasted_iota(jnp.int32, h.shape, 0)
          + (pl.program_id(0) + off) * be)
  return jnp.where(rows > 0, h, 0.0)


def _gate_mid_body(hun_ref, fzr_ref, fh_ref, wz2_ref, wh2_ref,
                   ur_ref, hu_ref, *, be, off=0):
  h = _gru_h(hun_ref, fzr_ref, fh_ref, wz2_ref, wh2_ref, be, off)
  hb = h.astype(jnp.bfloat16)
  u = jnp.dot(hb, ur_ref[...], preferred_element_type=jnp.float32)
  hu_ref[...] = _pack_hu(h, u)


def _gate_mid_alias_body(acc_ref, hun_ref, fzr_ref, fh_ref, wz2_ref,
                         wh2_ref, ur_ref, hu_ref, *, be, off=0):
  del acc_ref
  _gate_mid_body(hun_ref, fzr_ref, fh_ref, wz2_ref, wh2_ref,
                 ur_ref, hu_ref, be=be, off=off)


def _gate_last_body(hun_ref, fzr_ref, fh_ref, wz2_ref, wh2_ref,
                    h_ref, *, be, off=0):
  h_ref[...] = _gru_h(hun_ref, fzr_ref, fh_ref, wz2_ref, wh2_ref, be, off)


def _gate_last_alias_body(acc_ref, hun_ref, fzr_ref, fh_ref, wz2_ref,
                          wh2_ref, h_ref, *, be, off=0):
  del acc_ref
  _gate_last_body(hun_ref, fzr_ref, fh_ref, wz2_ref, wh2_ref,
                  h_ref, be=be, off=off)


def _out_body(hg_ref, fn_ref, msk_ref, wo1_ref, wo2_ref, bo_ref, o_ref):
  ns = jnp.sum(hg_ref[...], axis=0)
  o = jnp.dot(fn_ref[...], wo1_ref[...], preferred_element_type=jnp.float32)
  o += jnp.dot(ns, wo2_ref[...], preferred_element_type=jnp.float32)
  o += bo_ref[...]
  o_ref[...] = jnp.maximum(o, 0.0) * msk_ref[...]


def _out_alias_body(acc_ref, hg_ref, fn_ref, msk_ref, wo1_ref, wo2_ref,
                    bo_ref, o_ref):
  del acc_ref
  _out_body(hg_ref, fn_ref, msk_ref, wo1_ref, wo2_ref, bo_ref, o_ref)


def _row_spec(b, h, n_extra_lead=0):
  if n_extra_lead:
    return pl.BlockSpec((n_extra_lead, b, h), lambda i: (0, i, 0))
  return pl.BlockSpec((b, h), lambda i: (i, 0))


def _full_spec(shape):
  nd = len(shape)
  return pl.BlockSpec(shape, lambda i: (0,) * nd)


def kernel(fnode, fmess, agraph, bgraph, mask,
           Wz_w, Wz_b, Wr_w, Ur_w, Ur_b, Wh_w, Wh_b, Wo_w, Wo_b):
  n, fdim = fnode.shape
  e, h = fmess.shape
  nb = bgraph.shape[1]
  depth = 3

  Wz1, Wz2 = Wz_w[:h], Wz_w[h:]
  Wh1, Wh2 = Wh_w[:h], Wh_w[h:]
  Wo1, Wo2 = Wo_w[:fdim], Wo_w[fdim:]
  bz = Wz_b.reshape(1, h)
  bh = Wh_b.reshape(1, h)
  br = Ur_b.reshape(1, h)
  bo = Wo_b.reshape(1, h)

  n_slab = 5
  es = e // n_slab
  bidx_s = [bgraph[s * es:(s + 1) * es].T.reshape(-1) for s in range(n_slab)]
  npad = 10240
  ag = jnp.pad(agraph, ((0, npad - n), (0, 0)))
  aidx = ag.T.reshape(-1)
  fnode_p = jnp.pad(fnode, ((0, npad - n), (0, 0)))
  mask_p = jnp.pad(mask, ((0, npad - n), (0, 0)))

  be_pre = 3200
  g_pre = e // be_pre
  pre = pl.pallas_call(
      functools.partial(_pre_body, be=be_pre),
      grid=(g_pre,),
      in_specs=[
          _row_spec(be_pre, h),
          _full_spec((h, h)), _full_spec((h, h)), _full_spec((h, h)),
          _full_spec((h, h)),
          _full_spec((1, h)), _full_spec((1, h)), _full_spec((1, h)),
      ],
      out_specs=[_row_spec(be_pre, h)] * 3,
      out_shape=[jax.ShapeDtypeStruct((e, h), jnp.int32),
                 jax.ShapeDtypeStruct((e, h), jnp.float32),
                 jax.ShapeDtypeStruct((e, h), jnp.int32)],
      compiler_params=pltpu.CompilerParams(
          dimension_semantics=("arbitrary",)),
  )
  fzr, fhb, hu = pre(fmess, Wz1, Wh1, Wr_w, Ur_w, bz, bh, br)
  Wz2b = Wz2.astype(jnp.bfloat16)
  Wh2b = Wh2.astype(jnp.bfloat16)
  Urb = Ur_w.astype(jnp.bfloat16)

  sc_gather_hu = _make_sc_gather(nb * es, 120, h, jnp.int32)

  be_g = 3200
  blk_slab = es // be_g

  def _mk_gate(body, out_dtype, n_weights, slab):
    off = slab * blk_slab
    aliased = slab > 0
    slab_spec = pl.BlockSpec((be_g, h), lambda i, off=off: (off + i, 0))
    in_specs = [
        _row_spec(be_g, h, nb),
        slab_spec, slab_spec,
    ] + [_full_spec((h, h))] * n_weights
    if aliased:
      in_specs = [pl.BlockSpec(memory_space=pl.ANY)] + in_specs
    return pl.pallas_call(
        functools.partial(body, be=be_g, off=off),
        grid=(blk_slab,),
        in_specs=in_specs,
        out_specs=slab_spec,
        out_shape=jax.ShapeDtypeStruct((e, h), out_dtype),
        input_output_aliases={0: 0} if aliased else {},
        compiler_params=pltpu.CompilerParams(
            dimension_semantics=("arbitrary",)),
    )

  gate_mid = [_mk_gate(_gate_mid_body if s == 0 else _gate_mid_alias_body,
                       jnp.int32, 3, s) for s in range(n_slab)]
  gate_last = [_mk_gate(_gate_last_body if s == 0 else _gate_last_alias_body,
                        jnp.float32, 2, s) for s in range(n_slab)]

  for d in range(depth - 1):
    huns = [sc_gather_hu(bidx_s[s], hu).reshape(nb, es, h)
            for s in range(n_slab)]
    last = d == depth - 2
    if not last:
      acc = gate_mid[0](huns[0], fzr, fhb, Wz2b, Wh2b, Urb)
      for s in range(1, n_slab):
        acc = gate_mid[s](acc, huns[s], fzr, fhb, Wz2b, Wh2b, Urb)
      hu = acc
    else:
      acc = gate_last[0](huns[0], fzr, fhb, Wz2b, Wh2b)
      for s in range(1, n_slab):
        acc = gate_last[s](acc, huns[s], fzr, fhb, Wz2b, Wh2b)
      hcur = acc

  n_oslab = 2
  nps = npad // n_oslab
  aidx_s = [ag[s * nps:(s + 1) * nps].T.reshape(-1) for s in range(n_oslab)]
  sc_gather_h = _make_sc_gather(nb * nps, 120, h, jnp.float32)
  hgs = [sc_gather_h(aidx_s[s], hcur).reshape(nb, nps, h)
         for s in range(n_oslab)]

  bn = 1024
  blk_oslab = nps // bn

  def _mk_out(slab):
    off = slab * blk_oslab
    aliased = slab > 0
    in_specs = [
        _row_spec(bn, h, nb),
        pl.BlockSpec((bn, h), lambda i, off=off: (off + i, 0)),
        pl.BlockSpec((bn, 1), lambda i, off=off: (off + i, 0)),
        _full_spec((h, h)), _full_spec((h, h)), _full_spec((1, h)),
    ]
    if aliased:
      in_specs = [pl.BlockSpec(memory_space=pl.ANY)] + in_specs
    return pl.pallas_call(
        _out_alias_body if aliased else _out_body,
        grid=(blk_oslab,),
        in_specs=in_specs,
        out_specs=pl.BlockSpec((bn, h), lambda i, off=off: (off + i, 0)),
        out_shape=jax.ShapeDtypeStruct((npad, h), jnp.float32),
        input_output_aliases={0: 0} if aliased else {},
        compiler_params=pltpu.CompilerParams(
            dimension_semantics=("arbitrary",)),
    )

  acc = _mk_out(0)(hgs[0], fnode_p, mask_p, Wo1, Wo2, bo)
  for s in range(1, n_oslab):
    acc = _mk_out(s)(acc, hgs[s], fnode_p, mask_p, Wo1, Wo2, bo)
  return acc[:n], hcur
